# baseline (speedup 1.0000x reference)
import functools

import jax
import jax.numpy as jnp
from jax import lax
from jax.experimental import pallas as pl
from jax.experimental.pallas import tpu as pltpu
from jax.experimental.pallas import tpu_sc as plsc

_B = 16384
_F = 26
_V = 100000
_D = 16
_DEEP_IN = _F * _D
_H1 = 64
_H2 = 32

_NC = 2
_NS = 16
_NW = _NC * _NS
_NIDX = _B * _F
_NPW = _NIDX // _NW
_RPW = _NPW // 128
_RPG = 13
_GROUPS = _RPW // _RPG
_IPG = _RPG * 128


_TW = 512
_CPF = (_V - 160) // _TW
_VMAIN = _CPF * _TW
_NCH = _F * _CPF
_TAIL = _V - _VMAIN
_TAILSZ = _TAIL * _D


def _sc_transpose_body(tab_hbm, tail_hbm, out_hbm, in_v, out_v, tail_v):
    w = lax.axis_index("s") * _NC + lax.axis_index("c")
    iota = lax.iota(jnp.int32, 16)

    @pl.when(w < _F)
    def _():
        pltpu.sync_copy(tail_hbm.at[pl.ds(w * _TAILSZ, _TAILSZ)], tail_v)
        pltpu.sync_copy(tail_v,
                        out_hbm.at[pl.ds((w * _V + _VMAIN) * _D, _TAILSZ)])

    def body(k, carry):
        t = k * _NW + w

        @pl.when(t < _NCH)
        def _():
            f = t // _CPF
            v0 = (t % _CPF) * _TW
            pltpu.sync_copy(tab_hbm.at[f, :, pl.ds(v0, _TW)], in_v)
            for v in range(_TW):
                col = plsc.load_gather(
                    in_v, [iota, jnp.full((16,), v, jnp.int32)])
                out_v[pl.ds(v * _D, _D)] = col
            pltpu.sync_copy(out_v,
                            out_hbm.at[pl.ds((f * _V + v0) * _D, _TW * _D)])
        return carry

    lax.fori_loop(0, (_NCH + _NW - 1) // _NW, body, 0)


def _sc_transpose(tab_dmajor, tail_flat):
    mesh = plsc.VectorSubcoreMesh(core_axis_name="c", subcore_axis_name="s")
    return pl.kernel(
        _sc_transpose_body,
        mesh=mesh,
        out_type=jax.ShapeDtypeStruct((_F * _V * _D,), jnp.float32),
        scratch_types=[
            pltpu.VMEM((_D, _TW), jnp.float32),
            pltpu.VMEM((_TW * _D,), jnp.float32),
            pltpu.VMEM((_TAILSZ,), jnp.float32),
        ],
        compiler_params=pltpu.CompilerParams(
            use_tc_tiling_on_sc=True, needs_layout_passes=False),
    )(tab_dmajor, tail_flat)


def _sc_gather_body(idx_hbm, table_hbm, out_hbm, idx_v, rows_v, sem):
    wid = lax.axis_index("s") * _NC + lax.axis_index("c")
    pltpu.sync_copy(idx_hbm.at[pl.ds(wid * _RPW, _RPW)], idx_v)
    out_base = wid * _NPW

    def group(g, carry):
        copies = []
        for j in range(_RPG):
            c = pltpu.async_copy(
                table_hbm.at[idx_v.at[g * _RPG + j]],
                rows_v.at[pl.ds(j * 128, 128)],
                sem,
            )
            copies.append(c)
        for c in copies:
            c.wait()
        pltpu.sync_copy(rows_v, out_hbm.at[pl.ds(out_base + g * _IPG, _IPG)])
        return carry

    lax.fori_loop(0, _GROUPS, group, 0)


def _sc_gather(idx2d, table_flat):
    mesh = plsc.VectorSubcoreMesh(core_axis_name="c", subcore_axis_name="s")
    return pl.kernel(
        _sc_gather_body,
        mesh=mesh,
        out_type=jax.ShapeDtypeStruct((_NIDX, _D), jnp.float32),
        scratch_types=[
            pltpu.VMEM((_RPW, 128), jnp.int32),
            pltpu.VMEM((_IPG, _D), jnp.float32),
            pltpu.SemaphoreType.DMA,
        ],
        compiler_params=pltpu.CompilerParams(use_tc_tiling_on_sc=False),
    )(idx2d, table_flat)


_BM = 2048


def _tc_body(x_ref, emb_ref, s_ref, wlin_ref, w1_ref, b1_ref, w2_ref,
             b2_ref, w3_ref, bias_ref, out_ref):
    e = emb_ref[...]
    xf = x_ref[...].astype(jnp.float32)
    lin = xf @ wlin_ref[...]
    s = jnp.dot(e, s_ref[...], preferred_element_type=jnp.float32)
    sq = jnp.sum(e * e, axis=1, keepdims=True)
    fm = 0.5 * (jnp.sum(s * s, axis=1, keepdims=True) - sq)
    h = jnp.dot(e, w1_ref[...], preferred_element_type=jnp.float32) + b1_ref[...]
    h = jnp.maximum(h, 0.0)
    h = jnp.dot(h, w2_ref[...], preferred_element_type=jnp.float32) + b2_ref[...]
    h = jnp.maximum(h, 0.0)
    deep = jnp.dot(h, w3_ref[...], preferred_element_type=jnp.float32)
    out_ref[...] = jax.nn.sigmoid(lin + fm + deep + bias_ref[0, 0])


def _tc_call(X, emb, s_mat, w_lin, w1f, b1f, w2f, b2f, w3, bias):
    grid = (_B // _BM,)
    return pl.pallas_call(
        _tc_body,
        grid=grid,
        in_specs=[
            pl.BlockSpec((_BM, _F), lambda i: (i, 0)),
            pl.BlockSpec((_BM, _DEEP_IN), lambda i: (i, 0)),
            pl.BlockSpec((_DEEP_IN, _D), lambda i: (0, 0)),
            pl.BlockSpec((_F, 1), lambda i: (0, 0)),
            pl.BlockSpec((_DEEP_IN, _H1), lambda i: (0, 0)),
            pl.BlockSpec((1, _H1), lambda i: (0, 0)),
            pl.BlockSpec((_H1, _H2), lambda i: (0, 0)),
            pl.BlockSpec((1, _H2), lambda i: (0, 0)),
            pl.BlockSpec((_H2, 1), lambda i: (0, 0)),
            pl.BlockSpec((1, 1), lambda i: (0, 0)),
        ],
        out_specs=pl.BlockSpec((_BM, 1), lambda i: (i, 0)),
        out_shape=jax.ShapeDtypeStruct((_B, 1), jnp.float32),
    )(X, emb, s_mat, w_lin, w1f, b1f, w2f, b2f, w3, bias)


def kernel(X, tables, W_lin, b_lin, W1, b1, g1, be1, W2, b2, g2, be2, W3, b3):
    eps = 1e-5
    inv = 1.0 / jnp.sqrt(1.0 + eps)
    s1 = g1 * inv
    w1f = W1 * s1[None, :]
    b1f = (b1 * s1 + be1)[None, :]
    s2 = g2 * inv
    w2f = W2 * s2[None, :]
    b2f = (b2 * s2 + be2)[None, :]
    idx = (X + jnp.arange(_F, dtype=jnp.int32)[None, :] * _V).reshape(
        _NIDX // 128, 128)
    tab_dmajor = jnp.transpose(tables, (0, 2, 1))
    tail_flat = tables[:, _VMAIN:, :].reshape(_F * _TAILSZ)
    table_flat = _sc_transpose(tab_dmajor, tail_flat).reshape(_F * _V, _D)
    emb = _sc_gather(idx, table_flat).reshape(_B, _DEEP_IN)
    s_mat = jnp.tile(jnp.eye(_D, dtype=jnp.float32), (_F, 1))
    bias = (b_lin + b3).reshape(1, 1)
    return _tc_call(X, emb, s_mat, W_lin, w1f, b1f, w2f, b2f, W3, bias)


# speedup vs baseline: 6.1031x; 6.1031x over previous
import functools

import jax
import jax.numpy as jnp
from jax import lax
from jax.experimental import pallas as pl
from jax.experimental.pallas import tpu as pltpu
from jax.experimental.pallas import tpu_sc as plsc

_B = 16384
_F = 26
_V = 100000
_D = 16
_DEEP_IN = _F * _D
_H1 = 64
_H2 = 32

_NC = 2
_NS = 16
_NW = _NC * _NS
_NIDX = _B * _F
_NPW = _NIDX // _NW
_RPW = _NPW // 128
_RPG = 13
_GROUPS = _RPW // _RPG
_IPG = _RPG * 128


_TW = 512
_CPF = (_V - 160) // _TW
_VMAIN = _CPF * _TW
_NCH = _F * _CPF
_TAIL = _V - _VMAIN
_TAILSZ = _TAIL * _D
_NBUF = 4


def _sc_transpose_body(tab_hbm, tail_hbm, out_hbm,
                       in_v0, in_v1, in_v2, in_v3,
                       out_v0, out_v1, out_v2, out_v3, tail_v,
                       sin0, sin1, sin2, sin3,
                       sout0, sout1, sout2, sout3):
    w = lax.axis_index("s") * _NC + lax.axis_index("c")
    iota = lax.iota(jnp.int32, 16)
    in_bufs = (in_v0, in_v1, in_v2, in_v3)
    out_bufs = (out_v0, out_v1, out_v2, out_v3)
    sins = (sin0, sin1, sin2, sin3)
    souts = (sout0, sout1, sout2, sout3)

    @pl.when(w < _F)
    def _():
        pltpu.sync_copy(tail_hbm.at[pl.ds(w * _TAILSZ, _TAILSZ)], tail_v)
        pltpu.sync_copy(tail_v,
                        out_hbm.at[pl.ds((w * _V + _VMAIN) * _D, _TAILSZ)])

    def valid(k):
        return k * _NW + w < _NCH

    def in_copy(b, k):
        t = k * _NW + w
        f = t // _CPF
        v0 = (t % _CPF) * _TW
        return pltpu.make_async_copy(
            tab_hbm.at[f, :, pl.ds(v0, _TW)], in_bufs[b], sins[b])

    def out_copy(b, k):
        t = k * _NW + w
        f = t // _CPF
        v0 = (t % _CPF) * _TW
        return pltpu.make_async_copy(
            out_bufs[b], out_hbm.at[pl.ds((f * _V + v0) * _D, _TW * _D)],
            souts[b])

    for b in range(_NBUF):
        in_copy(b, b).start()

    def outer(kq, carry):
        for b in range(_NBUF):
            k = kq * _NBUF + b

            @pl.when(valid(k))
            def _(b=b, k=k):
                @pl.when(kq > 0)
                def _():
                    out_copy(b, k - _NBUF).wait()
                in_copy(b, k).wait()

                @plsc.parallel_loop(0, _TW, 1, unroll=8)
                def _(v):
                    vv = jnp.bitwise_and(v + iota, _TW - 1)
                    col = plsc.load_gather(in_bufs[b], [iota, vv])
                    plsc.store_scatter(out_bufs[b], [vv * _D + iota], col)

                @pl.when(valid(k + _NBUF))
                def _():
                    in_copy(b, k + _NBUF).start()
                out_copy(b, k).start()
        return carry

    nk = (_NCH + _NW - 1) // _NW
    lax.fori_loop(0, (nk + _NBUF - 1) // _NBUF, outer, 0)
    for b in range(_NBUF):
        out_copy(b, b).wait()


def _sc_transpose(tab_dmajor, tail_flat):
    mesh = plsc.VectorSubcoreMesh(core_axis_name="c", subcore_axis_name="s")
    return pl.kernel(
        _sc_transpose_body,
        mesh=mesh,
        out_type=jax.ShapeDtypeStruct((_F * _V * _D,), jnp.float32),
        scratch_types=(
            [pltpu.VMEM((_D, _TW), jnp.float32)] * _NBUF
            + [pltpu.VMEM((_TW * _D,), jnp.float32)] * _NBUF
            + [pltpu.VMEM((_TAILSZ,), jnp.float32)]
            + [pltpu.SemaphoreType.DMA] * (2 * _NBUF)
        ),
        compiler_params=pltpu.CompilerParams(
            use_tc_tiling_on_sc=True, needs_layout_passes=False),
    )(tab_dmajor, tail_flat)


def _sc_gather_body(idx_hbm, table_hbm, out_hbm, idx_v, rows_v, sem):
    wid = lax.axis_index("s") * _NC + lax.axis_index("c")
    pltpu.sync_copy(idx_hbm.at[pl.ds(wid * _RPW, _RPW)], idx_v)
    out_base = wid * _NPW

    def group(g, carry):
        copies = []
        for j in range(_RPG):
            c = pltpu.async_copy(
                table_hbm.at[idx_v.at[g * _RPG + j]],
                rows_v.at[pl.ds(j * 128, 128)],
                sem,
            )
            copies.append(c)
        for c in copies:
            c.wait()
        pltpu.sync_copy(rows_v, out_hbm.at[pl.ds(out_base + g * _IPG, _IPG)])
        return carry

    lax.fori_loop(0, _GROUPS, group, 0)


def _sc_gather(idx2d, table_flat):
    mesh = plsc.VectorSubcoreMesh(core_axis_name="c", subcore_axis_name="s")
    return pl.kernel(
        _sc_gather_body,
        mesh=mesh,
        out_type=jax.ShapeDtypeStruct((_NIDX, _D), jnp.float32),
        scratch_types=[
            pltpu.VMEM((_RPW, 128), jnp.int32),
            pltpu.VMEM((_IPG, _D), jnp.float32),
            pltpu.SemaphoreType.DMA,
        ],
        compiler_params=pltpu.CompilerParams(use_tc_tiling_on_sc=False),
    )(idx2d, table_flat)


_BM = 2048


def _tc_body(x_ref, emb_ref, s_ref, wlin_ref, w1_ref, b1_ref, w2_ref,
             b2_ref, w3_ref, bias_ref, out_ref):
    e = emb_ref[...]
    xf = x_ref[...].astype(jnp.float32)
    lin = xf @ wlin_ref[...]
    s = jnp.dot(e, s_ref[...], preferred_element_type=jnp.float32)
    sq = jnp.sum(e * e, axis=1, keepdims=True)
    fm = 0.5 * (jnp.sum(s * s, axis=1, keepdims=True) - sq)
    h = jnp.dot(e, w1_ref[...], preferred_element_type=jnp.float32) + b1_ref[...]
    h = jnp.maximum(h, 0.0)
    h = jnp.dot(h, w2_ref[...], preferred_element_type=jnp.float32) + b2_ref[...]
    h = jnp.maximum(h, 0.0)
    deep = jnp.dot(h, w3_ref[...], preferred_element_type=jnp.float32)
    out_ref[...] = jax.nn.sigmoid(lin + fm + deep + bias_ref[0, 0])


def _tc_call(X, emb, s_mat, w_lin, w1f, b1f, w2f, b2f, w3, bias):
    grid = (_B // _BM,)
    return pl.pallas_call(
        _tc_body,
        grid=grid,
        in_specs=[
            pl.BlockSpec((_BM, _F), lambda i: (i, 0)),
            pl.BlockSpec((_BM, _DEEP_IN), lambda i: (i, 0)),
            pl.BlockSpec((_DEEP_IN, _D), lambda i: (0, 0)),
            pl.BlockSpec((_F, 1), lambda i: (0, 0)),
            pl.BlockSpec((_DEEP_IN, _H1), lambda i: (0, 0)),
            pl.BlockSpec((1, _H1), lambda i: (0, 0)),
            pl.BlockSpec((_H1, _H2), lambda i: (0, 0)),
            pl.BlockSpec((1, _H2), lambda i: (0, 0)),
            pl.BlockSpec((_H2, 1), lambda i: (0, 0)),
            pl.BlockSpec((1, 1), lambda i: (0, 0)),
        ],
        out_specs=pl.BlockSpec((_BM, 1), lambda i: (i, 0)),
        out_shape=jax.ShapeDtypeStruct((_B, 1), jnp.float32),
    )(X, emb, s_mat, w_lin, w1f, b1f, w2f, b2f, w3, bias)


def kernel(X, tables, W_lin, b_lin, W1, b1, g1, be1, W2, b2, g2, be2, W3, b3):
    eps = 1e-5
    inv = 1.0 / jnp.sqrt(1.0 + eps)
    s1 = g1 * inv
    w1f = W1 * s1[None, :]
    b1f = (b1 * s1 + be1)[None, :]
    s2 = g2 * inv
    w2f = W2 * s2[None, :]
    b2f = (b2 * s2 + be2)[None, :]
    idx = (X + jnp.arange(_F, dtype=jnp.int32)[None, :] * _V).reshape(
        _NIDX // 128, 128)
    tab_dmajor = jnp.transpose(tables, (0, 2, 1))
    tail_flat = tables[:, _VMAIN:, :].reshape(_F * _TAILSZ)
    table_flat = _sc_transpose(tab_dmajor, tail_flat).reshape(_F * _V, _D)
    emb = _sc_gather(idx, table_flat).reshape(_B, _DEEP_IN)
    s_mat = jnp.tile(jnp.eye(_D, dtype=jnp.float32), (_F, 1))
    bias = (b_lin + b3).reshape(1, 1)
    return _tc_call(X, emb, s_mat, W_lin, w1f, b1f, w2f, b2f, W3, bias)
